# baseline (speedup 1.0000x reference)
import jax
import jax.numpy as jnp
from jax.experimental import pallas as pl

_NCLS = 81
_A = 20000
_B = 16
_C = 2000
_NC = _A // _C
_NS = 4
_CS = _C // _NS
_ALPHA = 0.2


def _ce_kernel(cls0_ref, cls1_ref, cls2_ref, cls3_ref, tgt_ref, tgtg_ref,
               lp_ref, lt_ref, ce_ref, npos_ref, posce_ref, sl1_ref):
    nc = pl.program_id(1)
    ces = []
    for ref in (cls0_ref, cls1_ref, cls2_ref, cls3_ref):
        xt = ref[0, 0, 0].T
        m = jnp.max(xt, axis=0, keepdims=True)
        s = jnp.sum(jnp.exp(xt - m), axis=0, keepdims=True)
        ces.append((m, s, xt))
    tgt = tgt_ref[0, 0]
    cls_iota = jax.lax.broadcasted_iota(jnp.int32, (_NCLS, _CS), 0)
    ce_parts = []
    for i, (m, s, xt) in enumerate(ces):
        tgt_i = tgt[:, i * _CS:(i + 1) * _CS]
        picked = jnp.sum(jnp.where(cls_iota == tgt_i, xt, 0.0), axis=0,
                         keepdims=True)
        ce_parts.append(m + jnp.log(s) - picked)
    ce = jnp.concatenate(ce_parts, axis=1)
    ce_ref[0, 0] = ce

    pos = tgt > 0
    posf = pos.astype(jnp.float32)
    npos_c = jnp.sum(pos.astype(jnp.int32)).reshape(1, 1)
    posce_c = jnp.sum(ce * posf).reshape(1, 1)

    d = lp_ref[0, 0] - lt_ref[0, 0]
    ad = jnp.abs(d)
    sl1 = jnp.where(ad < 1.0, 0.5 * d * d, ad - 0.5)
    gsel = (jax.lax.broadcasted_iota(jnp.int32, (128, 16), 0) // 8 ==
            jax.lax.broadcasted_iota(jnp.int32, (128, 16), 1)
            ).astype(jnp.float32)
    s_anchor = jax.lax.dot(sl1, gsel,
                           precision=jax.lax.Precision.HIGHEST)
    maskg = (tgtg_ref[0, 0] > 0).astype(jnp.float32)
    sl1_c = jnp.sum(s_anchor * maskg).reshape(1, 1)

    @pl.when(nc == 0)
    def _():
        npos_ref[0] = npos_c
        posce_ref[0] = posce_c
        sl1_ref[0] = sl1_c

    @pl.when(nc != 0)
    def _():
        npos_ref[0] += npos_c
        posce_ref[0] += posce_c
        sl1_ref[0] += sl1_c


def _mine_kernel(ce_ref, tgt_ref, npos_ref, top_ref):
    ce = jnp.maximum(ce_ref[:], 0.0)
    tgt = tgt_ref[:]
    np_ = npos_ref[:]
    total = jnp.sum(np_)
    m_neg = _A - np_
    k = jnp.where(total > 0, 3 * np_, 10)
    kp = jnp.minimum(k, m_neg)

    keys = jnp.where(tgt == 0, jax.lax.bitcast_convert_type(ce, jnp.int32), 0)

    def body(i, carry):
        prefix, rem = carry
        bit = 30 - i
        q = jnp.right_shift(prefix, bit) | 1
        matches = jnp.right_shift(keys, bit) == q
        cnt = jnp.sum(matches.astype(jnp.int32), axis=1, keepdims=True)
        take = rem <= cnt
        prefix = jnp.where(take, prefix | jnp.left_shift(1, bit), prefix)
        rem = jnp.where(take, rem, rem - cnt)
        return prefix, rem

    prefix0 = jnp.zeros((_B, 1), jnp.int32)
    rem0 = jnp.maximum(kp, 1)
    prefix, _ = jax.lax.fori_loop(0, 31, body, (prefix0, rem0))

    tval = jax.lax.bitcast_convert_type(prefix, jnp.float32)
    gt = keys > prefix
    cnt_gt = jnp.sum(gt.astype(jnp.int32), axis=1, keepdims=True)
    sum_gt = jnp.sum(jnp.where(gt, ce, 0.0), axis=1, keepdims=True)
    top = sum_gt + (kp - cnt_gt).astype(jnp.float32) * tval
    top_ref[:] = jnp.where(kp >= 1, top, 0.0)


def kernel(loc_preds, loc_targets, cls_preds, cls_targets):
    tgt = cls_targets.astype(jnp.int32)
    tgt4 = tgt.reshape(_B, _NC, 1, _C)
    _R = _C * 8 // 128
    loc_p = loc_preds.reshape(_B, _NC, _R, 128)
    loc_t = loc_targets.reshape(_B, _NC, _R, 128)
    tgt_g = tgt.reshape(_B, _NC, _R, 16)
    cls5 = cls_preds.reshape(_B, _NC, _NS, _CS, _NCLS)
    cls_spec = [
        pl.BlockSpec((1, 1, 1, _CS, _NCLS),
                     lambda b, nc, s=s: (b, nc, s, 0, 0))
        for s in range(_NS)
    ]
    ce4, npos, posce, sl1 = pl.pallas_call(
        _ce_kernel,
        grid=(_B, _NC),
        in_specs=cls_spec + [
            pl.BlockSpec((1, 1, 1, _C), lambda b, nc: (b, nc, 0, 0)),
            pl.BlockSpec((1, 1, _R, 16), lambda b, nc: (b, nc, 0, 0)),
            pl.BlockSpec((1, 1, _R, 128), lambda b, nc: (b, nc, 0, 0)),
            pl.BlockSpec((1, 1, _R, 128), lambda b, nc: (b, nc, 0, 0)),
        ],
        out_specs=[
            pl.BlockSpec((1, 1, 1, _C), lambda b, nc: (b, nc, 0, 0)),
            pl.BlockSpec((1, 1, 1), lambda b, nc: (b, 0, 0)),
            pl.BlockSpec((1, 1, 1), lambda b, nc: (b, 0, 0)),
            pl.BlockSpec((1, 1, 1), lambda b, nc: (b, 0, 0)),
        ],
        out_shape=[
            jax.ShapeDtypeStruct((_B, _NC, 1, _C), jnp.float32),
            jax.ShapeDtypeStruct((_B, 1, 1), jnp.int32),
            jax.ShapeDtypeStruct((_B, 1, 1), jnp.float32),
            jax.ShapeDtypeStruct((_B, 1, 1), jnp.float32),
        ],
    )(cls5, cls5, cls5, cls5, tgt4, tgt_g, loc_p, loc_t)

    top = pl.pallas_call(
        _mine_kernel,
        out_shape=jax.ShapeDtypeStruct((_B, 1), jnp.float32),
    )(ce4.reshape(_B, _A), tgt, npos.reshape(_B, 1))

    np_b = npos[:, 0, 0]
    total = np_b.sum()
    loc_loss = sl1.sum()
    cls_sum = posce.sum() + top.sum()
    nneg_b = jnp.where(total > 0, jnp.minimum(3 * np_b, _A), 10)
    nnt = nneg_b.sum()
    loss = jnp.where(
        total > 0,
        (_ALPHA * loc_loss + cls_sum) / (total + nnt).astype(jnp.float32),
        cls_sum / nnt.astype(jnp.float32),
    )
    return loss


# speedup vs baseline: 2.1040x; 2.1040x over previous
import jax
import jax.numpy as jnp
from jax.experimental import pallas as pl

_NCLS = 81
_A = 20000
_B = 16
_C = 2000
_NC = _A // _C
_NS = 4
_CS = _C // _NS
_ALPHA = 0.2


def _ce_kernel(cls_ref, tgt_ref, tgtg_ref, lp_ref, lt_ref,
               ce_ref, npos_ref, posce_ref, sl1_ref):
    xt = cls_ref[0]
    m = jnp.max(xt, axis=0, keepdims=True)
    s = jnp.sum(jnp.exp(xt - m), axis=0, keepdims=True)
    tgt = tgt_ref[0]
    cls_iota = jax.lax.broadcasted_iota(jnp.int32, (_NCLS, _A), 0)
    picked = jnp.sum(jnp.where(cls_iota == tgt, xt, 0.0), axis=0,
                     keepdims=True)
    ce = m + jnp.log(s) - picked
    ce_ref[0] = ce

    pos = tgt > 0
    posf = pos.astype(jnp.float32)
    npos_ref[0] = jnp.sum(pos.astype(jnp.int32)).reshape(1, 1)
    posce_ref[0] = jnp.sum(ce * posf).reshape(1, 1)

    d = lp_ref[0] - lt_ref[0]
    ad = jnp.abs(d)
    sl1 = jnp.where(ad < 1.0, 0.5 * d * d, ad - 0.5)
    gsel = (jax.lax.broadcasted_iota(jnp.int32, (128, 16), 0) // 8 ==
            jax.lax.broadcasted_iota(jnp.int32, (128, 16), 1)
            ).astype(jnp.float32)
    s_anchor = jax.lax.dot(sl1, gsel,
                           precision=jax.lax.Precision.HIGHEST)
    maskg = (tgtg_ref[0] > 0).astype(jnp.float32)
    sl1_ref[0] = jnp.sum(s_anchor * maskg).reshape(1, 1)


def _mine_kernel(ce_ref, tgt_ref, npos_ref, top_ref):
    ce = jnp.maximum(ce_ref[:], 0.0)
    tgt = tgt_ref[:]
    np_ = npos_ref[:]
    total = jnp.sum(np_)
    m_neg = _A - np_
    k = jnp.where(total > 0, 3 * np_, 10)
    kp = jnp.minimum(k, m_neg)

    keys = jnp.where(tgt == 0, jax.lax.bitcast_convert_type(ce, jnp.int32), 0)

    def body(i, carry):
        prefix, rem = carry
        bit = 30 - i
        q = jnp.right_shift(prefix, bit) | 1
        matches = jnp.right_shift(keys, bit) == q
        cnt = jnp.sum(matches.astype(jnp.int32), axis=1, keepdims=True)
        take = rem <= cnt
        prefix = jnp.where(take, prefix | jnp.left_shift(1, bit), prefix)
        rem = jnp.where(take, rem, rem - cnt)
        return prefix, rem

    prefix0 = jnp.zeros((_B, 1), jnp.int32)
    rem0 = jnp.maximum(kp, 1)
    prefix, _ = jax.lax.fori_loop(0, 31, body, (prefix0, rem0))

    tval = jax.lax.bitcast_convert_type(prefix, jnp.float32)
    gt = keys > prefix
    cnt_gt = jnp.sum(gt.astype(jnp.int32), axis=1, keepdims=True)
    sum_gt = jnp.sum(jnp.where(gt, ce, 0.0), axis=1, keepdims=True)
    top = sum_gt + (kp - cnt_gt).astype(jnp.float32) * tval
    top_ref[:] = jnp.where(kp >= 1, top, 0.0)


def kernel(loc_preds, loc_targets, cls_preds, cls_targets):
    tgt = cls_targets.astype(jnp.int32)
    tgt2 = tgt.reshape(_B, 1, _A)
    _R = _A * 8 // 128
    loc_p = loc_preds.reshape(_B, _R, 128)
    loc_t = loc_targets.reshape(_B, _R, 128)
    tgt_g = tgt.reshape(_B, _R, 16)
    ce2, npos, posce, sl1 = pl.pallas_call(
        _ce_kernel,
        grid=(_B,),
        in_specs=[
            pl.BlockSpec((1, _NCLS, _A), lambda b: (b, 0, 0)),
            pl.BlockSpec((1, 1, _A), lambda b: (b, 0, 0)),
            pl.BlockSpec((1, _R, 16), lambda b: (b, 0, 0)),
            pl.BlockSpec((1, _R, 128), lambda b: (b, 0, 0)),
            pl.BlockSpec((1, _R, 128), lambda b: (b, 0, 0)),
        ],
        out_specs=[
            pl.BlockSpec((1, 1, _A), lambda b: (b, 0, 0)),
            pl.BlockSpec((1, 1, 1), lambda b: (b, 0, 0)),
            pl.BlockSpec((1, 1, 1), lambda b: (b, 0, 0)),
            pl.BlockSpec((1, 1, 1), lambda b: (b, 0, 0)),
        ],
        out_shape=[
            jax.ShapeDtypeStruct((_B, 1, _A), jnp.float32),
            jax.ShapeDtypeStruct((_B, 1, 1), jnp.int32),
            jax.ShapeDtypeStruct((_B, 1, 1), jnp.float32),
            jax.ShapeDtypeStruct((_B, 1, 1), jnp.float32),
        ],
    )(jnp.swapaxes(cls_preds, 1, 2), tgt2, tgt_g, loc_p, loc_t)

    top = pl.pallas_call(
        _mine_kernel,
        out_shape=jax.ShapeDtypeStruct((_B, 1), jnp.float32),
    )(ce2.reshape(_B, _A), tgt, npos.reshape(_B, 1))

    np_b = npos[:, 0, 0]
    total = np_b.sum()
    loc_loss = sl1.sum()
    cls_sum = posce.sum() + top.sum()
    nneg_b = jnp.where(total > 0, jnp.minimum(3 * np_b, _A), 10)
    nnt = nneg_b.sum()
    loss = jnp.where(
        total > 0,
        (_ALPHA * loc_loss + cls_sum) / (total + nnt).astype(jnp.float32),
        cls_sum / nnt.astype(jnp.float32),
    )
    return loss
